# baseline (speedup 1.0000x reference)
import jax
import jax.numpy as jnp
from jax.experimental import pallas as pl
from jax.experimental.pallas import tpu as pltpu

_B, _N, _D, _U = 4, 2048, 128, 128


def _mpnn_body(x_ref, adj_ref, wmsg_ref, wupd_ref, out_ref):
    xT = x_ref[0].astype(jnp.bfloat16).T
    a = adj_ref[0]
    wmT = wmsg_ref[...].astype(jnp.bfloat16).T
    wuT = wupd_ref[...].astype(jnp.bfloat16).T

    msgT = jax.lax.dot(wmT, xT, preferred_element_type=jnp.float32)

    lhs = jnp.concatenate(
        [msgT.astype(jnp.bfloat16), jnp.ones((16, _N), dtype=jnp.bfloat16)],
        axis=0)
    p = jax.lax.dot(lhs, a.astype(jnp.bfloat16),
                    preferred_element_type=jnp.float32)
    aggT = p[:_U]
    deg = p[_U:_U + 1]

    updT = jax.lax.dot(wuT, xT, preferred_element_type=jnp.float32)

    msgs = jnp.where(deg > 0, aggT / jnp.maximum(deg, 1.0), 0.0)
    out_ref[0] = (updT + msgs).T


def kernel(x, adj, W_msg, W_upd):
    adj = adj.astype(jnp.float4_e2m1fn)
    return pl.pallas_call(
        _mpnn_body,
        grid=(_B,),
        in_specs=[
            pl.BlockSpec((1, _N, _D), lambda b: (b, 0, 0)),
            pl.BlockSpec((1, _N, _N), lambda b: (b, 0, 0)),
            pl.BlockSpec((_D, _U), lambda b: (0, 0)),
            pl.BlockSpec((_D, _U), lambda b: (0, 0)),
        ],
        out_specs=pl.BlockSpec((1, _N, _U), lambda b: (b, 0, 0)),
        out_shape=jax.ShapeDtypeStruct((_B, _N, _U), jnp.float32),
    )(x, adj, W_msg, W_upd)


# speedup vs baseline: 1.2227x; 1.2227x over previous
import jax
import jax.numpy as jnp
from jax.experimental import pallas as pl
from jax.experimental.pallas import tpu as pltpu

_B, _N, _D, _U = 4, 2048, 128, 128
_RBLK = _N // 2


def _mpnn_body(x_ref, adj_ref, wmsg_ref, wupd_ref, out_ref):
    r = pl.program_id(1)
    xT = x_ref[0].astype(jnp.bfloat16).T
    a = adj_ref[0]
    wmT = wmsg_ref[...].astype(jnp.bfloat16).T
    wuT = wupd_ref[...].astype(jnp.bfloat16).T

    msgT = jax.lax.dot(wmT, xT, preferred_element_type=jnp.float32)
    lhs = jnp.concatenate(
        [msgT.astype(jnp.bfloat16), jnp.ones((16, _N), dtype=jnp.bfloat16)],
        axis=0)
    p = jax.lax.dot(lhs, a.astype(jnp.bfloat16),
                    preferred_element_type=jnp.float32)
    aggT = p[:_U]
    deg = p[_U:_U + 1]

    xTr = x_ref[0, pl.ds(r * _RBLK, _RBLK), :].astype(jnp.bfloat16).T
    updT = jax.lax.dot(wuT, xTr, preferred_element_type=jnp.float32)

    msgs = jnp.where(deg > 0, aggT / jnp.maximum(deg, 1.0), 0.0)
    out_ref[0] = (updT + msgs).T


def kernel(x, adj, W_msg, W_upd):
    adj = adj.astype(jnp.int4)
    return pl.pallas_call(
        _mpnn_body,
        grid=(_B, 2),
        in_specs=[
            pl.BlockSpec((1, _N, _D), lambda b, r: (b, 0, 0)),
            pl.BlockSpec((1, _N, _RBLK), lambda b, r: (b, 0, r)),
            pl.BlockSpec((_D, _U), lambda b, r: (0, 0)),
            pl.BlockSpec((_D, _U), lambda b, r: (0, 0)),
        ],
        out_specs=pl.BlockSpec((1, _RBLK, _U), lambda b, r: (b, r, 0)),
        out_shape=jax.ShapeDtypeStruct((_B, _N, _U), jnp.float32),
    )(x, adj, W_msg, W_upd)


# speedup vs baseline: 1.4055x; 1.1495x over previous
import jax
import jax.numpy as jnp
from jax.experimental import pallas as pl
from jax.experimental.pallas import tpu as pltpu

_B, _N, _D, _U = 4, 2048, 128, 128


def _mpnn_body(x_ref, adj_ref, wmsg_ref, wupd_ref, out_ref):
    xT = x_ref[0].astype(jnp.bfloat16).T
    a = adj_ref[0]
    wmT = wmsg_ref[...].astype(jnp.bfloat16).T
    wuT = wupd_ref[...].astype(jnp.bfloat16).T

    msgT = jax.lax.dot(wmT, xT, preferred_element_type=jnp.float32)

    lhs = jnp.concatenate(
        [msgT.astype(jnp.bfloat16), jnp.ones((16, _N), dtype=jnp.bfloat16)],
        axis=0)
    p = jax.lax.dot(lhs, a.astype(jnp.bfloat16),
                    preferred_element_type=jnp.float32)
    aggT = p[:_U]
    deg = p[_U:_U + 1]

    updT = jax.lax.dot(wuT, xT, preferred_element_type=jnp.float32)

    msgs = jnp.where(deg > 0, aggT / jnp.maximum(deg, 1.0), 0.0)
    out_ref[0] = (updT + msgs).T


def kernel(x, adj, W_msg, W_upd):
    adj = adj.astype(jnp.int4)
    return pl.pallas_call(
        _mpnn_body,
        grid=(_B,),
        in_specs=[
            pl.BlockSpec((1, _N, _D), lambda b: (b, 0, 0)),
            pl.BlockSpec((1, _N, _N), lambda b: (b, 0, 0)),
            pl.BlockSpec((_D, _U), lambda b: (0, 0)),
            pl.BlockSpec((_D, _U), lambda b: (0, 0)),
        ],
        out_specs=pl.BlockSpec((1, _N, _U), lambda b: (b, 0, 0)),
        out_shape=jax.ShapeDtypeStruct((_B, _N, _U), jnp.float32),
    )(x, adj, W_msg, W_upd)
